# baseline (speedup 1.0000x reference)
import functools

import jax
import jax.numpy as jnp
from jax import lax
from jax.experimental import pallas as pl
from jax.experimental.pallas import tpu as pltpu
from jax.experimental.pallas import tpu_sc as plsc

_VOCAB = 100000
_DIM = 128
_FUSE_ROWS = 400

_NC = 2
_NS = 16
_NW = _NC * _NS
_CHUNK = 128
_NBUF = 2


def _fuse_body(r_ref, i_ref, o_ref):
    r = r_ref[...]
    i = i_ref[...]
    o_ref[...] = jnp.stack([r, i], axis=-1).reshape(_FUSE_ROWS, 2 * _DIM)


def _fuse_tables(embed_real, embed_imag):
    return pl.pallas_call(
        _fuse_body,
        grid=(_VOCAB // _FUSE_ROWS,),
        in_specs=[
            pl.BlockSpec((_FUSE_ROWS, _DIM), lambda g: (g, 0)),
            pl.BlockSpec((_FUSE_ROWS, _DIM), lambda g: (g, 0)),
        ],
        out_specs=pl.BlockSpec((_FUSE_ROWS, 2 * _DIM), lambda g: (g, 0)),
        out_shape=jax.ShapeDtypeStruct((_VOCAB, 2 * _DIM), jnp.float32),
    )(embed_real, embed_imag)


def _gather_rows(fused, ids_flat):
    n = ids_flat.shape[0]
    per_w = n // _NW
    n_chunks = per_w // _CHUNK
    n_groups = n_chunks // _NBUF
    row_w = 2 * _DIM
    mesh = plsc.VectorSubcoreMesh(core_axis_name="c", subcore_axis_name="s")

    @functools.partial(
        pl.kernel,
        out_type=jax.ShapeDtypeStruct((n, row_w), jnp.float32),
        mesh=mesh,
        scratch_types=[
            pltpu.VMEM((_NBUF, _CHUNK), jnp.int32),
            pltpu.VMEM((_NBUF, _CHUNK, row_w), jnp.float32),
            pltpu.SemaphoreType.DMA,
            pltpu.SemaphoreType.DMA,
        ],
    )
    def k(fused_hbm, ids_hbm, out_hbm, idx_v, rows_v, sem_g, sem_w):
        wid = lax.axis_index("s") * _NC + lax.axis_index("c")
        base = wid * per_w

        def group(g, carry):
            gathers = []
            for b in range(_NBUF):
                start = base + (g * _NBUF + b) * _CHUNK

                @pl.when(g > 0)
                def _wait_prev_write():
                    pltpu.make_async_copy(
                        rows_v.at[b], out_hbm.at[pl.ds(0, _CHUNK)], sem_w
                    ).wait()

                pltpu.sync_copy(ids_hbm.at[pl.ds(start, _CHUNK)], idx_v.at[b])
                gathers.append(
                    pltpu.async_copy(fused_hbm.at[idx_v.at[b]], rows_v.at[b], sem_g)
                )
            for b in range(_NBUF):
                start = base + (g * _NBUF + b) * _CHUNK
                gathers[b].wait()
                pltpu.async_copy(rows_v.at[b], out_hbm.at[pl.ds(start, _CHUNK)], sem_w)
            return carry

        lax.fori_loop(0, n_groups, group, 0, unroll=False)
        for b in range(_NBUF):
            pltpu.make_async_copy(
                rows_v.at[b], out_hbm.at[pl.ds(0, _CHUNK)], sem_w
            ).wait()

    return k(fused, ids_flat)


def kernel(ids, embed_real, embed_imag):
    b, l = ids.shape
    fused = _fuse_tables(embed_real, embed_imag)
    ids_flat = ids.reshape(-1).astype(jnp.int32)
    rows = _gather_rows(fused, ids_flat)
    return rows.reshape(b, l, _DIM, 2)
